# baseline (speedup 1.0000x reference)
import jax
import jax.numpy as jnp
from jax.experimental import pallas as pl
from jax.experimental.pallas import tpu as pltpu

_D_EMB = 64
_BS = 2048


def _probe_kernel(e_ref, w_ref, b_ref, o_ref, acc_ref, sem_ref):
    batch = o_ref.shape[0]
    S = o_ref.shape[1]
    bs = acc_ref.shape[0]
    ns = S // bs
    acc_ref[...] = jnp.broadcast_to(b_ref[...], acc_ref.shape)
    copies = [
        pltpu.make_async_copy(
            acc_ref,
            o_ref.at[j, pl.ds(i * bs, bs), :],
            sem_ref.at[i, j],
        )
        for i in range(ns)
        for j in range(batch)
    ]
    for c in copies:
        c.start()
    for c in copies:
        c.wait()


def kernel(x, embedding, W, b):
    B, S, D = x.shape
    bs = _BS
    ns = S // bs
    b2 = b.reshape(1, D)
    return pl.pallas_call(
        _probe_kernel,
        grid=(1,),
        in_specs=[
            pl.BlockSpec((bs, _D_EMB), lambda i: (0, 0)),
            pl.BlockSpec((_D_EMB, D), lambda i: (0, 0)),
            pl.BlockSpec((1, D), lambda i: (0, 0)),
        ],
        out_specs=pl.BlockSpec(memory_space=pltpu.MemorySpace.HBM),
        out_shape=jax.ShapeDtypeStruct((B, S, D), jnp.float32),
        scratch_shapes=[
            pltpu.VMEM((bs, D), jnp.float32),
            pltpu.SemaphoreType.DMA((ns, B)),
        ],
    )(embedding, W, b2)


# speedup vs baseline: 1.0014x; 1.0014x over previous
import jax
import jax.numpy as jnp
from jax.experimental import pallas as pl
from jax.experimental.pallas import tpu as pltpu

_D_EMB = 64
_BS = 512
_NBUF = 4


def _copies(acc_ref, o_ref, sem_ref, step, bs, batch):
    buf = step % _NBUF
    return [
        pltpu.make_async_copy(
            acc_ref.at[buf],
            o_ref.at[j, pl.ds(step * bs, bs), :],
            sem_ref.at[buf, j],
        )
        for j in range(batch)
    ]


def _pos_block_kernel(e_ref, w_ref, b_ref, o_ref, acc_ref, sem_ref):
    i = pl.program_id(0)
    ns = pl.num_programs(0)
    batch = o_ref.shape[0]
    bs = e_ref.shape[0]
    p = i % _NBUF

    @pl.when(i >= _NBUF)
    def _():
        for c in _copies(acc_ref, o_ref, sem_ref, i - _NBUF, bs, batch):
            c.wait()

    acc_ref[p] = (
        jnp.dot(e_ref[...], w_ref[...], preferred_element_type=jnp.float32)
        + b_ref[...]
    )
    for c in _copies(acc_ref, o_ref, sem_ref, i, bs, batch):
        c.start()

    @pl.when(i == ns - 1)
    def _():
        for back in range(_NBUF - 1, -1, -1):
            step = i - back

            @pl.when(step >= 0)
            def _():
                for c in _copies(acc_ref, o_ref, sem_ref, step, bs, batch):
                    c.wait()


def kernel(x, embedding, W, b):
    B, S, D = x.shape
    bs = _BS
    ns = S // bs
    b2 = b.reshape(1, D)
    return pl.pallas_call(
        _pos_block_kernel,
        grid=(ns,),
        in_specs=[
            pl.BlockSpec((bs, _D_EMB), lambda i: (i, 0)),
            pl.BlockSpec((_D_EMB, D), lambda i: (0, 0)),
            pl.BlockSpec((1, D), lambda i: (0, 0)),
        ],
        out_specs=pl.BlockSpec(memory_space=pltpu.MemorySpace.HBM),
        out_shape=jax.ShapeDtypeStruct((B, S, D), jnp.float32),
        scratch_shapes=[
            pltpu.VMEM((_NBUF, bs, D), jnp.float32),
            pltpu.SemaphoreType.DMA((_NBUF, B)),
        ],
        compiler_params=pltpu.CompilerParams(
            dimension_semantics=("arbitrary",),
        ),
    )(embedding, W, b2)
